# Initial kernel scaffold; baseline (speedup 1.0000x reference)
import jax
import jax.numpy as jnp
from jax.experimental import pallas as pl


def kernel(x, weights, top_weights, top_experts, up_w, gate_w, down_w):
    raise NotImplementedError("write your pallas kernel here")


# speedup vs baseline: 1.5553x; 1.5553x over previous
import functools

import jax
import jax.numpy as jnp
from jax.experimental import pallas as pl

HIDDEN = 1024
FFN = 2048
E = 16
TOPK = 2
FB = 512
NF = FFN // FB


def _moe_kernel(x_ref, tw_ref, te_ref, up_ref, gate_ref, down_ref, out_ref):
    e = pl.program_id(0)
    f = pl.program_id(1)

    xf = x_ref[...]
    up = up_ref[0]
    gate = gate_ref[0]
    down = down_ref[0]

    x1 = jax.lax.dot_general(xf, up, (((1,), (1,)), ((), ())),
                             preferred_element_type=jnp.float32)
    x2 = jax.lax.dot_general(xf, gate, (((1,), (1,)), ((), ())),
                             preferred_element_type=jnp.float32)
    h = x1 * jax.nn.sigmoid(x1) * x2
    partial = jax.lax.dot_general(h, down, (((1,), (1,)), ((), ())),
                                  preferred_element_type=jnp.float32)

    mask = te_ref[...] == e
    w = jnp.sum(jnp.where(mask, tw_ref[...], 0.0), axis=1, keepdims=True)
    contrib = partial * w

    first = (e == 0) & (f == 0)

    @pl.when(first)
    def _():
        out_ref[...] = contrib

    @pl.when(jnp.logical_not(first))
    def _():
        out_ref[...] += contrib


@functools.partial(jax.jit, static_argnames=())
def kernel(x, weights, top_weights, top_experts, up_w, gate_w, down_w):
    bsz, q_len, hidden = x.shape
    tok = bsz * q_len
    xf = x.reshape(tok, hidden)
    te = top_experts.astype(jnp.int32)

    out = pl.pallas_call(
        _moe_kernel,
        grid=(E, NF),
        in_specs=[
            pl.BlockSpec((tok, hidden), lambda e, f: (0, 0)),
            pl.BlockSpec((tok, TOPK), lambda e, f: (0, 0)),
            pl.BlockSpec((tok, TOPK), lambda e, f: (0, 0)),
            pl.BlockSpec((1, FB, hidden), lambda e, f: (e, f, 0)),
            pl.BlockSpec((1, FB, hidden), lambda e, f: (e, f, 0)),
            pl.BlockSpec((1, hidden, FB), lambda e, f: (e, 0, f)),
        ],
        out_specs=pl.BlockSpec((tok, hidden), lambda e, f: (0, 0)),
        out_shape=jax.ShapeDtypeStruct((tok, hidden), jnp.float32),
    )(xf, top_weights, te, up_w, gate_w, down_w)

    return out.reshape(bsz, q_len, hidden)


# speedup vs baseline: 1.6889x; 1.0859x over previous
import functools

import jax
import jax.numpy as jnp
from jax.experimental import pallas as pl

HIDDEN = 1024
FFN = 2048
E = 16
TOPK = 2
FB = 2048
NF = FFN // FB


def _moe_kernel(x_ref, tw_ref, te_ref, up_ref, gate_ref, down_ref, out_ref):
    e = pl.program_id(0)
    f = pl.program_id(1)

    xf = x_ref[...]
    up = up_ref[0]
    gate = gate_ref[0]
    down = down_ref[0]

    x1 = jax.lax.dot_general(xf, up, (((1,), (1,)), ((), ())),
                             preferred_element_type=jnp.float32)
    x2 = jax.lax.dot_general(xf, gate, (((1,), (1,)), ((), ())),
                             preferred_element_type=jnp.float32)
    h = x1 * jax.nn.sigmoid(x1) * x2
    partial = jax.lax.dot_general(h, down, (((1,), (1,)), ((), ())),
                                  preferred_element_type=jnp.float32)

    mask = te_ref[...] == e
    w = jnp.sum(jnp.where(mask, tw_ref[...], 0.0), axis=1, keepdims=True)
    contrib = partial * w

    first = (e == 0) & (f == 0)

    @pl.when(first)
    def _():
        out_ref[...] = contrib

    @pl.when(jnp.logical_not(first))
    def _():
        out_ref[...] += contrib


@functools.partial(jax.jit, static_argnames=())
def kernel(x, weights, top_weights, top_experts, up_w, gate_w, down_w):
    bsz, q_len, hidden = x.shape
    tok = bsz * q_len
    xf = x.reshape(tok, hidden)
    te = top_experts.astype(jnp.int32)

    out = pl.pallas_call(
        _moe_kernel,
        grid=(E, NF),
        in_specs=[
            pl.BlockSpec((tok, hidden), lambda e, f: (0, 0)),
            pl.BlockSpec((tok, TOPK), lambda e, f: (0, 0)),
            pl.BlockSpec((tok, TOPK), lambda e, f: (0, 0)),
            pl.BlockSpec((1, FB, hidden), lambda e, f: (e, f, 0)),
            pl.BlockSpec((1, FB, hidden), lambda e, f: (e, f, 0)),
            pl.BlockSpec((1, hidden, FB), lambda e, f: (e, 0, f)),
        ],
        out_specs=pl.BlockSpec((tok, hidden), lambda e, f: (0, 0)),
        out_shape=jax.ShapeDtypeStruct((tok, hidden), jnp.float32),
    )(xf, top_weights, te, up_w, gate_w, down_w)

    return out.reshape(bsz, q_len, hidden)


# speedup vs baseline: 1.7627x; 1.0437x over previous
import functools

import jax
import jax.numpy as jnp
from jax.experimental import pallas as pl

HIDDEN = 1024
FFN = 2048
E = 16
TOPK = 2
FB = 1024
NF = FFN // FB


def _moe_kernel(x_ref, tw_ref, te_ref, up_ref, gate_ref, down_ref, out_ref):
    e = pl.program_id(0)
    f = pl.program_id(1)

    xf = x_ref[...]
    up = up_ref[0]
    gate = gate_ref[0]
    down = down_ref[0]

    x1 = jax.lax.dot_general(xf, up, (((1,), (1,)), ((), ())),
                             preferred_element_type=jnp.float32)
    x2 = jax.lax.dot_general(xf, gate, (((1,), (1,)), ((), ())),
                             preferred_element_type=jnp.float32)
    h = x1 * jax.nn.sigmoid(x1) * x2
    partial = jax.lax.dot_general(h, down, (((1,), (1,)), ((), ())),
                                  preferred_element_type=jnp.float32)

    mask = te_ref[...] == e
    w = jnp.sum(jnp.where(mask, tw_ref[...], 0.0), axis=1, keepdims=True)
    contrib = partial * w

    first = (e == 0) & (f == 0)

    @pl.when(first)
    def _():
        out_ref[...] = contrib

    @pl.when(jnp.logical_not(first))
    def _():
        out_ref[...] += contrib


@functools.partial(jax.jit, static_argnames=())
def kernel(x, weights, top_weights, top_experts, up_w, gate_w, down_w):
    bsz, q_len, hidden = x.shape
    tok = bsz * q_len
    xf = x.reshape(tok, hidden)
    te = top_experts.astype(jnp.int32)

    out = pl.pallas_call(
        _moe_kernel,
        grid=(E, NF),
        in_specs=[
            pl.BlockSpec((tok, hidden), lambda e, f: (0, 0)),
            pl.BlockSpec((tok, TOPK), lambda e, f: (0, 0)),
            pl.BlockSpec((tok, TOPK), lambda e, f: (0, 0)),
            pl.BlockSpec((1, FB, hidden), lambda e, f: (e, f, 0)),
            pl.BlockSpec((1, FB, hidden), lambda e, f: (e, f, 0)),
            pl.BlockSpec((1, hidden, FB), lambda e, f: (e, 0, f)),
        ],
        out_specs=pl.BlockSpec((tok, hidden), lambda e, f: (0, 0)),
        out_shape=jax.ShapeDtypeStruct((tok, hidden), jnp.float32),
    )(xf, top_weights, te, up_w, gate_w, down_w)

    return out.reshape(bsz, q_len, hidden)
